# baseline (speedup 1.0000x reference)
import functools

import jax
import jax.numpy as jnp
from jax import lax
from jax.experimental import pallas as pl
from jax.experimental.pallas import tpu as pltpu
from jax.experimental.pallas import tpu_sc as plsc

N = 10000
IN = 128
HID = 16
HEADS = 8
OUT = 64
D1 = HEADS * HID

NC = 2
NS = 16
NW = NC * NS
L = 16

NP = 10112
RPT = NP // NS
E_TOT = 320000 + N

B1, B2 = 80, 80


def _nblk(b):
    return 4 * (-(-E_TOT // (4 * NW * b)))


NBLK1 = _nblk(B1)
NBLK2 = _nblk(B2)
EP_ARR = max(NBLK1 * NW * B1 + 2 * B1, NBLK2 * NW * B2 + 2 * B2)
BN = 1000


def _tc1_body(x_ref, w1_ref, as_ref, ad_ref, h_ref, a_s_ref, a_d_ref):
    h = jnp.dot(x_ref[...], w1_ref[...], preferred_element_type=jnp.float32)
    h_ref[...] = h
    a_s_ref[...] = jnp.dot(h, as_ref[...], preferred_element_type=jnp.float32)
    a_d_ref[...] = jnp.dot(h, ad_ref[...], preferred_element_type=jnp.float32)


def _tc1(x, W1, AS16, AD16):
    return pl.pallas_call(
        _tc1_body,
        grid=(N // BN,),
        in_specs=[
            pl.BlockSpec((BN, IN), lambda i: (i, 0)),
            pl.BlockSpec((IN, D1), lambda i: (0, 0)),
            pl.BlockSpec((D1, 16), lambda i: (0, 0)),
            pl.BlockSpec((D1, 16), lambda i: (0, 0)),
        ],
        out_specs=[
            pl.BlockSpec((BN, D1), lambda i: (i, 0)),
            pl.BlockSpec((BN, 16), lambda i: (i, 0)),
            pl.BlockSpec((BN, 16), lambda i: (i, 0)),
        ],
        out_shape=[
            jax.ShapeDtypeStruct((NP, D1), jnp.float32),
            jax.ShapeDtypeStruct((NP, 16), jnp.float32),
            jax.ShapeDtypeStruct((NP, 16), jnp.float32),
        ],
    )(x, W1, AS16, AD16)


def _tc2_body(acc_ref, den_ref, r_ref, b1_ref, w2_ref,
              ps_ref, pd_ref, h2_ref, a_s_ref, a_d_ref):
    den = den_ref[0] + den_ref[1]
    dfull = jnp.dot(den, r_ref[...], preferred_element_type=jnp.float32)
    g = (acc_ref[0] + acc_ref[1]) / (dfull + 1e-16) + b1_ref[...]
    hcur = jnp.where(g > 0.0, g, jnp.exp(g) - 1.0)
    h2 = jnp.dot(hcur, w2_ref[...], preferred_element_type=jnp.float32)
    h2_ref[...] = h2
    a_s_ref[...] = jnp.dot(h2, ps_ref[...], preferred_element_type=jnp.float32)
    a_d_ref[...] = jnp.dot(h2, pd_ref[...], preferred_element_type=jnp.float32)


def _tc2(acc, den, R, b1, W2, PS, PD):
    return pl.pallas_call(
        _tc2_body,
        grid=(N // BN,),
        in_specs=[
            pl.BlockSpec((NC, BN, D1), lambda i: (0, i, 0)),
            pl.BlockSpec((NC, BN, 16), lambda i: (0, i, 0)),
            pl.BlockSpec((16, D1), lambda i: (0, 0)),
            pl.BlockSpec((1, D1), lambda i: (0, 0)),
            pl.BlockSpec((D1, OUT), lambda i: (0, 0)),
            pl.BlockSpec((OUT, 16), lambda i: (0, 0)),
            pl.BlockSpec((OUT, 16), lambda i: (0, 0)),
        ],
        out_specs=[
            pl.BlockSpec((BN, OUT), lambda i: (i, 0)),
            pl.BlockSpec((BN, 16), lambda i: (i, 0)),
            pl.BlockSpec((BN, 16), lambda i: (i, 0)),
        ],
        out_shape=[
            jax.ShapeDtypeStruct((NP, OUT), jnp.float32),
            jax.ShapeDtypeStruct((NP, 16), jnp.float32),
            jax.ShapeDtypeStruct((NP, 16), jnp.float32),
        ],
    )(acc, den, R, b1, W2, PS, PD)


def _tc3_body(acc_ref, den_ref, q_ref, b2_ref, out_ref):
    den = jnp.dot(den_ref[0] + den_ref[1], q_ref[...],
                  preferred_element_type=jnp.float32)
    t = (acc_ref[0] + acc_ref[1]) / (den + 1e-16) + b2_ref[...]
    m = jnp.max(t, axis=1, keepdims=True)
    ex = jnp.exp(t - m)
    lse = jnp.log(jnp.sum(ex, axis=1, keepdims=True))
    out_ref[...] = t - m - lse


def _tc3(acc, den, Q, b2):
    return pl.pallas_call(
        _tc3_body,
        grid=(N // BN,),
        in_specs=[
            pl.BlockSpec((NC, BN, OUT), lambda i: (0, i, 0)),
            pl.BlockSpec((NC, BN, 16), lambda i: (0, i, 0)),
            pl.BlockSpec((16, OUT), lambda i: (0, 0)),
            pl.BlockSpec((1, OUT), lambda i: (0, 0)),
        ],
        out_specs=pl.BlockSpec((BN, OUT), lambda i: (i, 0)),
        out_shape=jax.ShapeDtypeStruct((N, OUT), jnp.float32),
    )(acc, den, Q, b2)


def _make_sc_edge(D, H, B, NBLK, name):
    CH = D // H
    mesh = plsc.VectorSubcoreMesh(
        core_axis_name="c", subcore_axis_name="s",
        num_cores=NC, num_subcores=NS)

    def body(h_hbm, as_hbm, ad_hbm, src_hbm, dst_hbm, zD_hbm, z16_hbm,
             acc_out, den_out, *rest):
        sidx = rest[0:4]
        didx = rest[4:8]
        gs = rest[8:10]
        gd = rest[10:12]
        hb = rest[12:14]
        exb = rest[14:16]
        acc_sh, den_sh = rest[16:18]
        gsem = (rest[18:21], rest[21:24])
        isem = rest[24:28]

        c = lax.axis_index("c")
        s = lax.axis_index("s")
        r0 = s * RPT
        wid = c * NS + s
        base0 = wid * (NBLK * B)
        lane = lax.broadcasted_iota(jnp.int32, (L,), 0)

        def idx_issue(b, q):
            base = base0 + b * B
            pltpu.async_copy(src_hbm.at[pl.ds(base, B)], sidx[q], isem[q])
            pltpu.async_copy(dst_hbm.at[pl.ds(base, B)], didx[q], isem[q])

        def idx_wait(b, q):
            base = base0 + b * B
            pltpu.make_async_copy(src_hbm.at[pl.ds(base, B)], sidx[q], isem[q]).wait()
            pltpu.make_async_copy(dst_hbm.at[pl.ds(base, B)], didx[q], isem[q]).wait()

        def g_issue(d, q):
            pltpu.async_copy(as_hbm.at[sidx[q]], gs[d], gsem[d][0])
            pltpu.async_copy(ad_hbm.at[didx[q]], gd[d], gsem[d][1])
            pltpu.async_copy(h_hbm.at[sidx[q]], hb[d], gsem[d][2])

        def g_wait(d, q):
            pltpu.make_async_copy(as_hbm.at[sidx[q]], gs[d], gsem[d][0]).wait()
            pltpu.make_async_copy(ad_hbm.at[didx[q]], gd[d], gsem[d][1]).wait()
            pltpu.make_async_copy(h_hbm.at[sidx[q]], hb[d], gsem[d][2]).wait()

        def compute(d):
            gs_d, gd_d, hb_d, exb_d = gs[d], gd[d], hb[d], exb[d]

            @plsc.parallel_loop(0, B, unroll=4)
            def edge(e):
                u = gs_d[e, :] + gd_d[e, :]
                a = jnp.where(u >= 0.0, u, 0.2 * u)
                exm = jnp.where(lane < H, jnp.exp(a), 0.0)
                exb_d[e, :] = exm
                for hd in range(H):
                    scv = jnp.full((L,), exm[hd], dtype=jnp.float32)
                    for v in range(CH // L):
                        col = hd * CH + v * L
                        hb_d[e, pl.ds(col, L)] = hb_d[e, pl.ds(col, L)] * scv

        def do_block(b, d, q, qn, qnn):
            g_wait(d, q)
            idx_wait(b + 1, qn)
            g_issue((d + 1) % 2, qn)
            idx_issue(b + 2, qnn)
            compute(d)
            pltpu.sync_copy(exb[d], den_sh.at[didx[q]], add=True)
            pltpu.sync_copy(hb[d], acc_sh.at[didx[q]], add=True)

        idx_issue(0, 0)
        idx_issue(1, 1)
        idx_wait(0, 0)
        g_issue(0, 0)
        pltpu.sync_copy(zD_hbm.at[pl.ds(r0, RPT)], acc_sh.at[pl.ds(r0, RPT)])
        pltpu.sync_copy(z16_hbm.at[pl.ds(r0, RPT)], den_sh.at[pl.ds(r0, RPT)])
        plsc.subcore_barrier()

        def quad(bb, carry):
            for p in range(4):
                b = bb * 4 + p
                do_block(b, p % 2, p, (p + 1) % 4, (p + 2) % 4)
            return carry

        lax.fori_loop(0, NBLK // 4, quad, 0)
        g_wait(0, 0)
        idx_wait(NBLK + 1, 1)
        plsc.subcore_barrier()
        pltpu.sync_copy(acc_sh.at[pl.ds(r0, RPT)], acc_out.at[c, pl.ds(r0, RPT)])
        pltpu.sync_copy(den_sh.at[pl.ds(r0, RPT)], den_out.at[c, pl.ds(r0, RPT)])

    return pl.kernel(
        body,
        out_type=(jax.ShapeDtypeStruct((NC, NP, D), jnp.float32),
                  jax.ShapeDtypeStruct((NC, NP, 16), jnp.float32)),
        mesh=mesh,
        scratch_types=(
            [pltpu.VMEM((B,), jnp.int32) for _ in range(8)]
            + [pltpu.VMEM((B, 16), jnp.float32) for _ in range(4)]
            + [pltpu.VMEM((B, D), jnp.float32) for _ in range(2)]
            + [pltpu.VMEM((B, 16), jnp.float32) for _ in range(2)]
            + [pltpu.VMEM_SHARED((NP, D), jnp.float32),
               pltpu.VMEM_SHARED((NP, 16), jnp.float32)]
            + [pltpu.SemaphoreType.DMA for _ in range(10)]
        ),
        compiler_params=pltpu.CompilerParams(use_tc_tiling_on_sc=False),
        name=name,
    )


_sc_edge1 = _make_sc_edge(D1, HEADS, B1, NBLK1, "gat_edge_l1")
_sc_edge2 = _make_sc_edge(OUT, 1, B2, NBLK2, "gat_edge_l2")


def kernel(x, edge_index, W1, att_src1, att_dst1, bias1,
           W2, att_src2, att_dst2, bias2):
    f32 = jnp.float32
    loop = jnp.arange(N, dtype=jnp.int32)
    padv = N + jnp.arange(EP_ARR - E_TOT, dtype=jnp.int32) % (NP - N)
    src = jnp.concatenate([edge_index[0], loop, padv])
    dst = jnp.concatenate([edge_index[1], loop, padv])

    af_s = att_src1.reshape(-1)
    af_d = att_dst1.reshape(-1)
    colh = jnp.arange(16)[None, :]
    rowh = (jnp.arange(D1) // HID)[:, None]
    AS16 = jnp.where(colh == rowh, af_s[:, None], 0.0).astype(f32)
    AD16 = jnp.where(colh == rowh, af_d[:, None], 0.0).astype(f32)
    R = jnp.where((jnp.arange(D1)[None, :] // HID) == jnp.arange(16)[:, None],
                  1.0, 0.0).astype(f32)
    PS = jnp.where(colh[:, :16] == 0, att_src2.reshape(-1)[:, None], 0.0).astype(f32)
    PD = jnp.where(colh[:, :16] == 0, att_dst2.reshape(-1)[:, None], 0.0).astype(f32)
    Q = jnp.where(jnp.arange(16)[:, None] == 0, jnp.ones((16, OUT), f32), 0.0)

    zD1 = jnp.zeros((NP, D1), f32)
    zD2 = jnp.zeros((NP, OUT), f32)
    z16 = jnp.zeros((NP, 16), f32)

    h1, a_s1, a_d1 = _tc1(x, W1, AS16, AD16)
    acc1, den1 = _sc_edge1(h1, a_s1, a_d1, src, dst, zD1, z16)

    h2, a_s2, a_d2 = _tc2(acc1, den1, R, bias1.reshape(1, D1), W2, PS, PD)
    acc2, den2 = _sc_edge2(h2, a_s2, a_d2, src, dst, zD2, z16)

    return _tc3(acc2, den2, Q, bias2.reshape(1, OUT))


# speedup vs baseline: 1.0630x; 1.0630x over previous
import functools

import jax
import jax.numpy as jnp
from jax import lax
from jax.experimental import pallas as pl
from jax.experimental.pallas import tpu as pltpu
from jax.experimental.pallas import tpu_sc as plsc

N = 10000
IN = 128
HID = 16
HEADS = 8
OUT = 64
D1 = HEADS * HID

NC = 2
NS = 16
NW = NC * NS
L = 16

NP = 10112
RPT = NP // NS
E_TOT = 320000 + N

B1, B2 = 96, 96


def _nblk(b):
    return 4 * (-(-E_TOT // (4 * NW * b)))


NBLK1 = _nblk(B1)
NBLK2 = _nblk(B2)
EP_ARR = max(NBLK1 * NW * B1 + 2 * B1, NBLK2 * NW * B2 + 2 * B2)
BN = 1000


def _tc1_body(x_ref, w1_ref, as_ref, ad_ref, h_ref, a_s_ref, a_d_ref):
    h = jnp.dot(x_ref[...], w1_ref[...], preferred_element_type=jnp.float32)
    h_ref[...] = h
    a_s_ref[...] = jnp.dot(h, as_ref[...], preferred_element_type=jnp.float32)
    a_d_ref[...] = jnp.dot(h, ad_ref[...], preferred_element_type=jnp.float32)


def _tc1(x, W1, AS16, AD16):
    return pl.pallas_call(
        _tc1_body,
        grid=(N // BN,),
        in_specs=[
            pl.BlockSpec((BN, IN), lambda i: (i, 0)),
            pl.BlockSpec((IN, D1), lambda i: (0, 0)),
            pl.BlockSpec((D1, 16), lambda i: (0, 0)),
            pl.BlockSpec((D1, 16), lambda i: (0, 0)),
        ],
        out_specs=[
            pl.BlockSpec((BN, D1), lambda i: (i, 0)),
            pl.BlockSpec((BN, 16), lambda i: (i, 0)),
            pl.BlockSpec((BN, 16), lambda i: (i, 0)),
        ],
        out_shape=[
            jax.ShapeDtypeStruct((NP, D1), jnp.float32),
            jax.ShapeDtypeStruct((NP, 16), jnp.float32),
            jax.ShapeDtypeStruct((NP, 16), jnp.float32),
        ],
    )(x, W1, AS16, AD16)


def _tc2_body(acc_ref, den_ref, r_ref, b1_ref, w2_ref,
              ps_ref, pd_ref, h2_ref, a_s_ref, a_d_ref):
    den = den_ref[0] + den_ref[1]
    dfull = jnp.dot(den, r_ref[...], preferred_element_type=jnp.float32)
    g = (acc_ref[0] + acc_ref[1]) / (dfull + 1e-16) + b1_ref[...]
    hcur = jnp.where(g > 0.0, g, jnp.exp(g) - 1.0)
    h2 = jnp.dot(hcur, w2_ref[...], preferred_element_type=jnp.float32)
    h2_ref[...] = h2
    a_s_ref[...] = jnp.dot(h2, ps_ref[...], preferred_element_type=jnp.float32)
    a_d_ref[...] = jnp.dot(h2, pd_ref[...], preferred_element_type=jnp.float32)


def _tc2(acc, den, R, b1, W2, PS, PD):
    return pl.pallas_call(
        _tc2_body,
        grid=(N // BN,),
        in_specs=[
            pl.BlockSpec((NC, BN, D1), lambda i: (0, i, 0)),
            pl.BlockSpec((NC, BN, 16), lambda i: (0, i, 0)),
            pl.BlockSpec((16, D1), lambda i: (0, 0)),
            pl.BlockSpec((1, D1), lambda i: (0, 0)),
            pl.BlockSpec((D1, OUT), lambda i: (0, 0)),
            pl.BlockSpec((OUT, 16), lambda i: (0, 0)),
            pl.BlockSpec((OUT, 16), lambda i: (0, 0)),
        ],
        out_specs=[
            pl.BlockSpec((BN, OUT), lambda i: (i, 0)),
            pl.BlockSpec((BN, 16), lambda i: (i, 0)),
            pl.BlockSpec((BN, 16), lambda i: (i, 0)),
        ],
        out_shape=[
            jax.ShapeDtypeStruct((NP, OUT), jnp.float32),
            jax.ShapeDtypeStruct((NP, 16), jnp.float32),
            jax.ShapeDtypeStruct((NP, 16), jnp.float32),
        ],
    )(acc, den, R, b1, W2, PS, PD)


def _tc3_body(acc_ref, den_ref, q_ref, b2_ref, out_ref):
    den = jnp.dot(den_ref[0] + den_ref[1], q_ref[...],
                  preferred_element_type=jnp.float32)
    t = (acc_ref[0] + acc_ref[1]) / (den + 1e-16) + b2_ref[...]
    m = jnp.max(t, axis=1, keepdims=True)
    ex = jnp.exp(t - m)
    lse = jnp.log(jnp.sum(ex, axis=1, keepdims=True))
    out_ref[...] = t - m - lse


def _tc3(acc, den, Q, b2):
    return pl.pallas_call(
        _tc3_body,
        grid=(N // BN,),
        in_specs=[
            pl.BlockSpec((NC, BN, OUT), lambda i: (0, i, 0)),
            pl.BlockSpec((NC, BN, 16), lambda i: (0, i, 0)),
            pl.BlockSpec((16, OUT), lambda i: (0, 0)),
            pl.BlockSpec((1, OUT), lambda i: (0, 0)),
        ],
        out_specs=pl.BlockSpec((BN, OUT), lambda i: (i, 0)),
        out_shape=jax.ShapeDtypeStruct((N, OUT), jnp.float32),
    )(acc, den, Q, b2)


def _make_sc_edge(D, H, B, NBLK, name):
    CH = D // H
    mesh = plsc.VectorSubcoreMesh(
        core_axis_name="c", subcore_axis_name="s",
        num_cores=NC, num_subcores=NS)

    def body(h_hbm, as_hbm, ad_hbm, src_hbm, dst_hbm, zD_hbm, z16_hbm,
             acc_out, den_out, *rest):
        sidx = rest[0:4]
        didx = rest[4:8]
        gs = rest[8:10]
        gd = rest[10:12]
        hb = rest[12:14]
        exb = rest[14:16]
        acc_sh, den_sh = rest[16:18]
        gsem = (rest[18:21], rest[21:24])
        isem = rest[24:28]

        c = lax.axis_index("c")
        s = lax.axis_index("s")
        r0 = s * RPT
        wid = c * NS + s
        base0 = wid * (NBLK * B)
        lane = lax.broadcasted_iota(jnp.int32, (L,), 0)

        def idx_issue(b, q):
            base = base0 + b * B
            pltpu.async_copy(src_hbm.at[pl.ds(base, B)], sidx[q], isem[q])
            pltpu.async_copy(dst_hbm.at[pl.ds(base, B)], didx[q], isem[q])

        def idx_wait(b, q):
            base = base0 + b * B
            pltpu.make_async_copy(src_hbm.at[pl.ds(base, B)], sidx[q], isem[q]).wait()
            pltpu.make_async_copy(dst_hbm.at[pl.ds(base, B)], didx[q], isem[q]).wait()

        def g_issue(d, q):
            pltpu.async_copy(as_hbm.at[sidx[q]], gs[d], gsem[d][0])
            pltpu.async_copy(ad_hbm.at[didx[q]], gd[d], gsem[d][1])
            pltpu.async_copy(h_hbm.at[sidx[q]], hb[d], gsem[d][2])

        def g_wait(d, q):
            pltpu.make_async_copy(as_hbm.at[sidx[q]], gs[d], gsem[d][0]).wait()
            pltpu.make_async_copy(ad_hbm.at[didx[q]], gd[d], gsem[d][1]).wait()
            pltpu.make_async_copy(h_hbm.at[sidx[q]], hb[d], gsem[d][2]).wait()

        def compute(d):
            gs_d, gd_d, hb_d, exb_d = gs[d], gd[d], hb[d], exb[d]

            @plsc.parallel_loop(0, B, unroll=4)
            def edge(e):
                u = gs_d[e, :] + gd_d[e, :]
                a = jnp.where(u >= 0.0, u, 0.2 * u)
                exm = jnp.where(lane < H, jnp.exp(a), 0.0)
                exb_d[e, :] = exm
                for hd in range(H):
                    scv = jnp.full((L,), exm[hd], dtype=jnp.float32)
                    for v in range(CH // L):
                        col = hd * CH + v * L
                        hb_d[e, pl.ds(col, L)] = hb_d[e, pl.ds(col, L)] * scv

        def do_block(b, d, q, qn, qnn):
            g_wait(d, q)
            idx_wait(b + 1, qn)
            g_issue((d + 1) % 2, qn)
            idx_issue(b + 2, qnn)
            compute(d)
            pltpu.sync_copy(exb[d], den_sh.at[didx[q]], add=True)
            pltpu.sync_copy(hb[d], acc_sh.at[didx[q]], add=True)

        idx_issue(0, 0)
        idx_issue(1, 1)
        idx_wait(0, 0)
        g_issue(0, 0)
        pltpu.sync_copy(zD_hbm.at[pl.ds(r0, RPT)], acc_sh.at[pl.ds(r0, RPT)])
        pltpu.sync_copy(z16_hbm.at[pl.ds(r0, RPT)], den_sh.at[pl.ds(r0, RPT)])
        plsc.subcore_barrier()

        def quad(bb, carry):
            for p in range(4):
                b = bb * 4 + p
                do_block(b, p % 2, p, (p + 1) % 4, (p + 2) % 4)
            return carry

        lax.fori_loop(0, NBLK // 4, quad, 0)
        g_wait(0, 0)
        idx_wait(NBLK + 1, 1)
        plsc.subcore_barrier()
        pltpu.sync_copy(acc_sh.at[pl.ds(r0, RPT)], acc_out.at[c, pl.ds(r0, RPT)])
        pltpu.sync_copy(den_sh.at[pl.ds(r0, RPT)], den_out.at[c, pl.ds(r0, RPT)])

    return pl.kernel(
        body,
        out_type=(jax.ShapeDtypeStruct((NC, NP, D), jnp.float32),
                  jax.ShapeDtypeStruct((NC, NP, 16), jnp.float32)),
        mesh=mesh,
        scratch_types=(
            [pltpu.VMEM((B,), jnp.int32) for _ in range(8)]
            + [pltpu.VMEM((B, 16), jnp.float32) for _ in range(4)]
            + [pltpu.VMEM((B, D), jnp.float32) for _ in range(2)]
            + [pltpu.VMEM((B, 16), jnp.float32) for _ in range(2)]
            + [pltpu.VMEM_SHARED((NP, D), jnp.float32),
               pltpu.VMEM_SHARED((NP, 16), jnp.float32)]
            + [pltpu.SemaphoreType.DMA for _ in range(10)]
        ),
        compiler_params=pltpu.CompilerParams(use_tc_tiling_on_sc=False),
        name=name,
    )


_sc_edge1 = _make_sc_edge(D1, HEADS, B1, NBLK1, "gat_edge_l1")
_sc_edge2 = _make_sc_edge(OUT, 1, B2, NBLK2, "gat_edge_l2")


def kernel(x, edge_index, W1, att_src1, att_dst1, bias1,
           W2, att_src2, att_dst2, bias2):
    f32 = jnp.float32
    loop = jnp.arange(N, dtype=jnp.int32)
    padv = N + jnp.arange(EP_ARR - E_TOT, dtype=jnp.int32) % (NP - N)
    src = jnp.concatenate([edge_index[0], loop, padv])
    dst = jnp.concatenate([edge_index[1], loop, padv])

    af_s = att_src1.reshape(-1)
    af_d = att_dst1.reshape(-1)
    colh = jnp.arange(16)[None, :]
    rowh = (jnp.arange(D1) // HID)[:, None]
    AS16 = jnp.where(colh == rowh, af_s[:, None], 0.0).astype(f32)
    AD16 = jnp.where(colh == rowh, af_d[:, None], 0.0).astype(f32)
    R = jnp.where((jnp.arange(D1)[None, :] // HID) == jnp.arange(16)[:, None],
                  1.0, 0.0).astype(f32)
    PS = jnp.where(colh[:, :16] == 0, att_src2.reshape(-1)[:, None], 0.0).astype(f32)
    PD = jnp.where(colh[:, :16] == 0, att_dst2.reshape(-1)[:, None], 0.0).astype(f32)
    Q = jnp.where(jnp.arange(16)[:, None] == 0, jnp.ones((16, OUT), f32), 0.0)

    zD1 = jnp.zeros((NP, D1), f32)
    zD2 = jnp.zeros((NP, OUT), f32)
    z16 = jnp.zeros((NP, 16), f32)

    h1, a_s1, a_d1 = _tc1(x, W1, AS16, AD16)
    acc1, den1 = _sc_edge1(h1, a_s1, a_d1, src, dst, zD1, z16)

    h2, a_s2, a_d2 = _tc2(acc1, den1, R, bias1.reshape(1, D1), W2, PS, PD)
    acc2, den2 = _sc_edge2(h2, a_s2, a_d2, src, dst, zD2, z16)

    return _tc3(acc2, den2, Q, bias2.reshape(1, OUT))
